# Initial kernel scaffold; baseline (speedup 1.0000x reference)
import jax
import jax.numpy as jnp
from jax.experimental import pallas as pl


def kernel(x, edge_index, W1, b1, W2, b2):
    raise NotImplementedError("write your pallas kernel here")


# speedup vs baseline: 22.0427x; 22.0427x over previous
import functools

import jax
import jax.numpy as jnp
from jax import lax
from jax.experimental import pallas as pl
from jax.experimental.pallas import tpu as pltpu
from jax.experimental.pallas import tpu_sc as plsc

N = 10000
E = 320000
D = 128

NC = 2
NS = 16
NW = NC * NS
N_PAD = 10240
ROWS_PER_TILE = N_PAD // NS
EDGES_PER_TILE = E // NW
CH = 125
NCH = EDGES_PER_TILE // CH

_mesh = plsc.VectorSubcoreMesh(core_axis_name="c", subcore_axis_name="s")


@functools.partial(
    pl.kernel,
    out_type=jax.ShapeDtypeStruct((NC, N_PAD), jnp.float32),
    mesh=_mesh,
    scratch_types=[
        pltpu.VMEM((NCH, CH), jnp.int32),
        pltpu.VMEM((CH,), jnp.float32),
        pltpu.VMEM_SHARED((N_PAD,), jnp.float32),
    ],
)
def _deg_kernel(rows_hbm, zeros_hbm, ones_hbm, deg_out, idx_v, ones_v, deg_sh):
    c = lax.axis_index("c")
    s = lax.axis_index("s")
    wid = c * NS + s
    pltpu.sync_copy(zeros_hbm, deg_sh.at[pl.ds(s * ROWS_PER_TILE, ROWS_PER_TILE)])
    pltpu.sync_copy(ones_hbm, ones_v)
    pltpu.sync_copy(rows_hbm.at[pl.ds(wid * NCH, NCH)], idx_v)
    plsc.subcore_barrier()

    def body(i, carry):
        pltpu.sync_copy(ones_v, deg_sh.at[idx_v.at[i]], add=True)
        return carry

    lax.fori_loop(0, NCH, body, 0)
    plsc.subcore_barrier()
    pltpu.sync_copy(
        deg_sh.at[pl.ds(s * ROWS_PER_TILE, ROWS_PER_TILE)],
        deg_out.at[c, pl.ds(s * ROWS_PER_TILE, ROWS_PER_TILE)],
    )


@functools.partial(
    pl.kernel,
    out_type=jax.ShapeDtypeStruct((NC, N_PAD, D), jnp.float32),
    mesh=_mesh,
    scratch_types=[
        pltpu.VMEM((NCH, CH), jnp.int32),
        pltpu.VMEM((NCH, CH), jnp.int32),
        pltpu.VMEM((CH, D), jnp.float32),
        pltpu.VMEM_SHARED((N_PAD, D), jnp.float32),
        pltpu.SemaphoreType.DMA,
    ],
)
def _scatter_kernel(g_hbm, rows_hbm, cols_hbm, zeros_hbm, acc_out,
                    rows_v, cols_v, gbuf, acc_sh, gsem):
    c = lax.axis_index("c")
    s = lax.axis_index("s")
    wid = c * NS + s
    pltpu.sync_copy(zeros_hbm, acc_sh.at[pl.ds(s * ROWS_PER_TILE, ROWS_PER_TILE)])
    pltpu.sync_copy(rows_hbm.at[pl.ds(wid * NCH, NCH)], rows_v)
    pltpu.sync_copy(cols_hbm.at[pl.ds(wid * NCH, NCH)], cols_v)
    plsc.subcore_barrier()

    def body(i, carry):
        pltpu.async_copy(g_hbm.at[cols_v.at[i]], gbuf, gsem).wait()
        pltpu.sync_copy(gbuf, acc_sh.at[rows_v.at[i]], add=True)
        return carry

    lax.fori_loop(0, NCH, body, 0)
    plsc.subcore_barrier()
    pltpu.sync_copy(
        acc_sh.at[pl.ds(s * ROWS_PER_TILE, ROWS_PER_TILE)],
        acc_out.at[c, pl.ds(s * ROWS_PER_TILE, ROWS_PER_TILE)],
    )


BLK = 1024
_GRID = N_PAD // BLK

_DN = (((1,), (1,)), ((), ()))


def _prep_body(deg_ref, x_ref, w_ref, b_ref, dinv_ref, g_ref):
    deg = deg_ref[...]
    d = deg[0] + deg[1] + 1.0
    dinv = lax.rsqrt(d)
    h = lax.dot_general(x_ref[...], w_ref[...], _DN,
                        preferred_element_type=jnp.float32) + b_ref[...]
    dinv_ref[...] = dinv
    g_ref[...] = dinv * h


def _mid_body(acc_ref, g1_ref, dinv_ref, w_ref, b_ref, g2_ref):
    acc = acc_ref[...]
    s = acc[0] + acc[1] + g1_ref[...]
    dinv = dinv_ref[...]
    h1 = jnp.maximum(dinv * s, 0.0)
    h = lax.dot_general(h1, w_ref[...], _DN,
                        preferred_element_type=jnp.float32) + b_ref[...]
    g2_ref[...] = dinv * h


def _fin_body(acc_ref, g2_ref, dinv_ref, out_ref):
    acc = acc_ref[...]
    out_ref[...] = dinv_ref[...] * (acc[0] + acc[1] + g2_ref[...])


_prep_call = pl.pallas_call(
    _prep_body,
    grid=(_GRID,),
    in_specs=[
        pl.BlockSpec((2, BLK, 1), lambda i: (0, i, 0)),
        pl.BlockSpec((BLK, D), lambda i: (i, 0)),
        pl.BlockSpec((D, D), lambda i: (0, 0)),
        pl.BlockSpec((1, D), lambda i: (0, 0)),
    ],
    out_specs=[
        pl.BlockSpec((BLK, 1), lambda i: (i, 0)),
        pl.BlockSpec((BLK, D), lambda i: (i, 0)),
    ],
    out_shape=[
        jax.ShapeDtypeStruct((N_PAD, 1), jnp.float32),
        jax.ShapeDtypeStruct((N_PAD, D), jnp.float32),
    ],
)

_mid_call = pl.pallas_call(
    _mid_body,
    grid=(_GRID,),
    in_specs=[
        pl.BlockSpec((2, BLK, D), lambda i: (0, i, 0)),
        pl.BlockSpec((BLK, D), lambda i: (i, 0)),
        pl.BlockSpec((BLK, 1), lambda i: (i, 0)),
        pl.BlockSpec((D, D), lambda i: (0, 0)),
        pl.BlockSpec((1, D), lambda i: (0, 0)),
    ],
    out_specs=pl.BlockSpec((BLK, D), lambda i: (i, 0)),
    out_shape=jax.ShapeDtypeStruct((N_PAD, D), jnp.float32),
)

_fin_call = pl.pallas_call(
    _fin_body,
    grid=(_GRID,),
    in_specs=[
        pl.BlockSpec((2, BLK, D), lambda i: (0, i, 0)),
        pl.BlockSpec((BLK, D), lambda i: (i, 0)),
        pl.BlockSpec((BLK, 1), lambda i: (i, 0)),
    ],
    out_specs=pl.BlockSpec((BLK, D), lambda i: (i, 0)),
    out_shape=jax.ShapeDtypeStruct((N_PAD, D), jnp.float32),
)


@jax.jit
def _run(x, edge_index, W1, b1, W2, b2):
    rows = edge_index[0].astype(jnp.int32).reshape(E // CH, CH)
    cols = edge_index[1].astype(jnp.int32).reshape(E // CH, CH)
    x_pad = jnp.pad(x, ((0, N_PAD - N), (0, 0)))

    zeros_row = jnp.zeros((ROWS_PER_TILE,), jnp.float32)
    ones_ch = jnp.ones((CH,), jnp.float32)
    zeros_blk = jnp.zeros((ROWS_PER_TILE, D), jnp.float32)

    deg = _deg_kernel(rows, zeros_row, ones_ch)
    deg3 = deg.reshape(NC, N_PAD, 1)

    dinv, g1 = _prep_call(deg3, x_pad, W1, b1.reshape(1, D))
    acc1 = _scatter_kernel(g1, rows, cols, zeros_blk)
    g2 = _mid_call(acc1, g1, dinv, W2, b2.reshape(1, D))
    acc2 = _scatter_kernel(g2, rows, cols, zeros_blk)
    out = _fin_call(acc2, g2, dinv)
    return out[:N]


def kernel(x, edge_index, W1, b1, W2, b2):
    return _run(x, edge_index, W1, b1, W2, b2)
